# baseline (speedup 1.0000x reference)
import jax
import jax.numpy as jnp
from jax import lax
from jax.experimental import pallas as pl


BATCHES_PER_STEP = 16


def _vq_body(xt_ref, wt_ref, idx_ref, qt_ref):
    wt = wt_ref[...]
    w2 = lax.transpose(jnp.sum(wt * wt, axis=0, keepdims=True), (1, 0))
    wt2 = wt + wt
    for j in range(BATCHES_PER_STEP):
        xt = xt_ref[j]
        x2 = jnp.sum(xt * xt, axis=0, keepdims=True)
        p2t = lax.dot_general(wt2, xt, (((0,), (0,)), ((), ())),
                              preferred_element_type=jnp.float32)
        d2 = (x2 + w2) - p2t
        dist = jnp.sqrt(jnp.maximum(d2, 0.0))
        k = dist.shape[0]
        m = jnp.min(dist, axis=0, keepdims=True)
        ksf = lax.broadcasted_iota(jnp.int32, dist.shape, 0).astype(jnp.float32)
        idxf = jnp.min(jnp.where(dist == m, ksf, float(k)), axis=0)
        idx_ref[j, 0] = idxf.astype(jnp.int32)
        onehot_t = (ksf == idxf[None, :]).astype(jnp.float32)
        qt_ref[j] = lax.dot_general(wt, onehot_t, (((1,), (0,)), ((), ())),
                                    preferred_element_type=jnp.float32)


def kernel(x, W):
    b, n, d = x.shape
    nk = W.shape[0]
    xt = x.transpose(0, 2, 1)
    wt = W.T
    g = BATCHES_PER_STEP
    idx, qt = pl.pallas_call(
        _vq_body,
        grid=(b // g,),
        in_specs=[
            pl.BlockSpec((g, d, n), lambda i: (i, 0, 0)),
            pl.BlockSpec((d, nk), lambda i: (0, 0)),
        ],
        out_specs=[
            pl.BlockSpec((g, 1, n), lambda i: (i, 0, 0)),
            pl.BlockSpec((g, d, n), lambda i: (i, 0, 0)),
        ],
        out_shape=[
            jax.ShapeDtypeStruct((b, 1, n), jnp.int32),
            jax.ShapeDtypeStruct((b, d, n), jnp.float32),
        ],
    )(xt, wt)
    return qt.transpose(0, 2, 1), idx.reshape(b, n)


# speedup vs baseline: 1.0179x; 1.0179x over previous
import jax
import jax.numpy as jnp
from jax import lax
from jax.experimental import pallas as pl


BATCHES_PER_STEP = 8


def _vq_body(xt_ref, wt_ref, idx_ref, qt_ref):
    wt = wt_ref[...]
    w2 = lax.transpose(jnp.sum(wt * wt, axis=0, keepdims=True), (1, 0))
    wt2 = wt + wt
    for j in range(BATCHES_PER_STEP):
        xt = xt_ref[j]
        x2 = jnp.sum(xt * xt, axis=0, keepdims=True)
        p2t = lax.dot_general(wt2, xt, (((0,), (0,)), ((), ())),
                              preferred_element_type=jnp.float32)
        d2 = (x2 + w2) - p2t
        dist = jnp.sqrt(jnp.maximum(d2, 0.0))
        k = dist.shape[0]
        m = jnp.min(dist, axis=0, keepdims=True)
        ksf = lax.broadcasted_iota(jnp.int32, dist.shape, 0).astype(jnp.float32)
        idxf = jnp.min(jnp.where(dist == m, ksf, float(k)), axis=0)
        idx_ref[j, 0] = idxf.astype(jnp.int32)
        onehot_t = (ksf == idxf[None, :]).astype(jnp.float32)
        qt_ref[j] = lax.dot_general(wt, onehot_t, (((1,), (0,)), ((), ())),
                                    preferred_element_type=jnp.float32)


def kernel(x, W):
    b, n, d = x.shape
    nk = W.shape[0]
    xt = x.transpose(0, 2, 1)
    wt = W.T
    g = BATCHES_PER_STEP
    idx, qt = pl.pallas_call(
        _vq_body,
        grid=(b // g,),
        in_specs=[
            pl.BlockSpec((g, d, n), lambda i: (i, 0, 0)),
            pl.BlockSpec((d, nk), lambda i: (0, 0)),
        ],
        out_specs=[
            pl.BlockSpec((g, 1, n), lambda i: (i, 0, 0)),
            pl.BlockSpec((g, d, n), lambda i: (i, 0, 0)),
        ],
        out_shape=[
            jax.ShapeDtypeStruct((b, 1, n), jnp.int32),
            jax.ShapeDtypeStruct((b, d, n), jnp.float32),
        ],
    )(xt, wt)
    return qt.transpose(0, 2, 1), idx.reshape(b, n)


# speedup vs baseline: 1.0568x; 1.0382x over previous
import jax
import jax.numpy as jnp
from jax import lax
from jax.experimental import pallas as pl


BATCHES_PER_STEP = 8


def _vq_body(xt_ref, wt_ref, idx_ref, qt_ref):
    wt = wt_ref[...]
    w2 = lax.transpose(jnp.sum(wt * wt, axis=0, keepdims=True), (1, 0))
    wt2 = wt + wt
    for j in range(BATCHES_PER_STEP):
        xt = xt_ref[j]
        x2 = jnp.sum(xt * xt, axis=0, keepdims=True)
        p2t = lax.dot_general(wt2, xt, (((0,), (0,)), ((), ())),
                              preferred_element_type=jnp.float32)
        d2 = (x2 + w2) - p2t
        mx = jnp.maximum(d2, 0.0)
        dist = jnp.sqrt(mx)
        m = jnp.sqrt(jnp.min(mx, axis=0, keepdims=True))
        k = dist.shape[0]
        ksf = lax.broadcasted_iota(jnp.int32, dist.shape, 0).astype(jnp.float32)
        idxf = jnp.min(jnp.where(dist == m, ksf, float(k)), axis=0)
        idx_ref[0, j] = idxf.astype(jnp.int32)
        onehot_t = (ksf == idxf[None, :]).astype(jnp.float32)
        qt_ref[j] = lax.dot_general(wt, onehot_t, (((1,), (0,)), ((), ())),
                                    preferred_element_type=jnp.float32)


def kernel(x, W):
    b, n, d = x.shape
    nk = W.shape[0]
    xt = x.transpose(0, 2, 1)
    wt = W.T
    g = BATCHES_PER_STEP
    idx, qt = pl.pallas_call(
        _vq_body,
        grid=(b // g,),
        in_specs=[
            pl.BlockSpec((g, d, n), lambda i: (i, 0, 0)),
            pl.BlockSpec((d, nk), lambda i: (0, 0)),
        ],
        out_specs=[
            pl.BlockSpec((1, g, n), lambda i: (i, 0, 0)),
            pl.BlockSpec((g, d, n), lambda i: (i, 0, 0)),
        ],
        out_shape=[
            jax.ShapeDtypeStruct((b // g, g, n), jnp.int32),
            jax.ShapeDtypeStruct((b, d, n), jnp.float32),
        ],
    )(xt, wt)
    return qt.transpose(0, 2, 1), idx.reshape(b, n)
